# Initial kernel scaffold; baseline (speedup 1.0000x reference)
import jax
import jax.numpy as jnp
from jax.experimental import pallas as pl


def kernel(x, edge_index, W1, b1, W2, b2):
    raise NotImplementedError("write your pallas kernel here")


# speedup vs baseline: 125.9799x; 125.9799x over previous
import functools

import jax
import jax.numpy as jnp
from jax import lax
from jax.experimental import pallas as pl
from jax.experimental.pallas import tpu as pltpu
from jax.experimental.pallas import tpu_sc as plsc

N_SC = 2
N_TILE = 16
EDGE_CHUNK = 5000


def _sc_mesh():
    return plsc.VectorSubcoreMesh(core_axis_name="c", subcore_axis_name="s")


_SC_PARAMS = pltpu.CompilerParams(use_tc_tiling_on_sc=False)


def _make_degree_kernel(n_pad, n_edges):
    per_tile = n_edges // (N_SC * N_TILE)
    n_chunks = per_tile // EDGE_CHUNK
    rows_pt = n_pad // N_TILE

    @functools.partial(
        pl.kernel,
        out_type=jax.ShapeDtypeStruct((N_SC, n_pad), jnp.float32),
        mesh=_sc_mesh(),
        compiler_params=_SC_PARAMS,
        scratch_types=[
            pltpu.VMEM_SHARED((n_pad,), jnp.float32),
            pltpu.VMEM((EDGE_CHUNK,), jnp.int32),
            pltpu.VMEM((EDGE_CHUNK,), jnp.float32),
        ],
    )
    def deg_kernel(dst_hbm, ones_hbm, zeros_hbm, out_hbm, accum, idx_d, ones_v):
        c = lax.axis_index("c")
        s = lax.axis_index("s")
        sl = pl.ds(s * rows_pt, rows_pt)
        pltpu.sync_copy(zeros_hbm.at[sl], accum.at[sl])
        pltpu.sync_copy(ones_hbm, ones_v)
        plsc.subcore_barrier()
        base = c * (n_edges // N_SC) + s * per_tile

        def body(i, carry):
            pltpu.sync_copy(dst_hbm.at[pl.ds(base + i * EDGE_CHUNK, EDGE_CHUNK)], idx_d)
            pltpu.sync_copy(ones_v, accum.at[idx_d], add=True)
            return carry

        lax.fori_loop(0, n_chunks, body, 0)
        plsc.subcore_barrier()
        pltpu.sync_copy(accum.at[sl], out_hbm.at[c, sl])

    return deg_kernel


def _make_prop_kernel(n_pad, n_edges, feat):
    per_tile = n_edges // (N_SC * N_TILE)
    n_chunks = per_tile // EDGE_CHUNK
    rows_pt = n_pad // N_TILE

    @functools.partial(
        pl.kernel,
        out_type=jax.ShapeDtypeStruct((N_SC, feat, n_pad), jnp.float32),
        mesh=_sc_mesh(),
        compiler_params=_SC_PARAMS,
        scratch_types=(
            [pltpu.VMEM_SHARED((n_pad,), jnp.float32) for _ in range(2 * feat)]
            + [pltpu.VMEM((EDGE_CHUNK,), jnp.int32)] * 2
            + [pltpu.VMEM((EDGE_CHUNK,), jnp.float32) for _ in range(feat)]
            + [pltpu.SemaphoreType.DMA]
        ),
    )
    def prop_kernel(yt_hbm, src_hbm, dst_hbm, zeros_hbm, out_hbm, *refs):
        tables = refs[:feat]
        accums = refs[feat:2 * feat]
        idx_s, idx_d = refs[2 * feat], refs[2 * feat + 1]
        rows = refs[2 * feat + 2: 3 * feat + 2]
        sem = refs[3 * feat + 2]
        c = lax.axis_index("c")
        s = lax.axis_index("s")
        sl = pl.ds(s * rows_pt, rows_pt)
        for k in range(feat):
            pltpu.sync_copy(yt_hbm.at[k, sl], tables[k].at[sl])
            pltpu.sync_copy(zeros_hbm.at[sl], accums[k].at[sl])
        plsc.subcore_barrier()
        base = c * (n_edges // N_SC) + s * per_tile

        def body(i, carry):
            e0 = base + i * EDGE_CHUNK
            pltpu.sync_copy(src_hbm.at[pl.ds(e0, EDGE_CHUNK)], idx_s)
            pltpu.sync_copy(dst_hbm.at[pl.ds(e0, EDGE_CHUNK)], idx_d)
            for k in range(feat):
                pltpu.async_copy(tables[k].at[idx_s], rows[k], sem).wait()
                pltpu.sync_copy(rows[k], accums[k].at[idx_d], add=True)
            return carry

        lax.fori_loop(0, n_chunks, body, 0)
        plsc.subcore_barrier()
        for k in range(feat):
            pltpu.sync_copy(accums[k].at[sl], out_hbm.at[c, k, sl])

    return prop_kernel


def _dense1_body(degp_ref, xt_ref, dis_ref, y1t_ref):
    deg = degp_ref[0:1, :] + degp_ref[1:2, :] + 1.0
    dis = lax.rsqrt(deg)
    dis_ref[...] = dis
    y1t_ref[...] = xt_ref[...] * dis


def _dense2_body(p_ref, y1t_ref, dis_ref, w1t_ref, b1_ref, w2t_ref, y2t_ref):
    dis = dis_ref[...]
    t = (p_ref[0] + p_ref[1] + y1t_ref[...]) * dis
    h = jnp.dot(w1t_ref[...], t, preferred_element_type=jnp.float32) + b1_ref[...]
    h = jnp.maximum(h, 0.0)
    y2t_ref[...] = jnp.dot(w2t_ref[...], h, preferred_element_type=jnp.float32) * dis


def _dense3_body(p_ref, y2t_ref, dis_ref, b2_ref, out_ref):
    out_ref[...] = (p_ref[0] + p_ref[1] + y2t_ref[...]) * dis_ref[...] + b2_ref[...]


def kernel(x, edge_index, W1, b1, W2, b2):
    n = x.shape[0]
    n_edges = edge_index.shape[1]
    blk = -(-n // (N_TILE * 128)) * 128
    n_pad = blk * N_TILE
    pad = n_pad - n

    ei = edge_index.astype(jnp.int32)
    src = ei[0]
    dst = ei[1]
    xt = jnp.concatenate([x, jnp.zeros((pad, x.shape[1]), x.dtype)], axis=0).T

    zeros = jnp.zeros((n_pad,), jnp.float32)
    ones = jnp.ones((EDGE_CHUNK,), jnp.float32)

    degp = _make_degree_kernel(n_pad, n_edges)(dst, ones, zeros)

    grid = (N_TILE,)
    fmaj = lambda f: pl.BlockSpec((f, blk), lambda i: (0, i))
    part = lambda f: pl.BlockSpec((N_SC, f, blk), lambda i: (0, 0, i))
    full = lambda a, b: pl.BlockSpec((a, b), lambda i: (0, 0))

    dis, y1t = pl.pallas_call(
        _dense1_body,
        grid=grid,
        in_specs=[pl.BlockSpec((N_SC, blk), lambda i: (0, i)), fmaj(4)],
        out_specs=[fmaj(1), fmaj(4)],
        out_shape=[
            jax.ShapeDtypeStruct((1, n_pad), jnp.float32),
            jax.ShapeDtypeStruct((4, n_pad), jnp.float32),
        ],
    )(degp, xt)

    p1 = _make_prop_kernel(n_pad, n_edges, 4)(y1t, src, dst, zeros)

    y2t = pl.pallas_call(
        _dense2_body,
        grid=grid,
        in_specs=[part(4), fmaj(4), fmaj(1),
                  full(16, 4), full(16, 1), full(2, 16)],
        out_specs=fmaj(2),
        out_shape=jax.ShapeDtypeStruct((2, n_pad), jnp.float32),
    )(p1, y1t, dis, W1.T, b1.reshape(16, 1), W2.T)

    p2 = _make_prop_kernel(n_pad, n_edges, 2)(y2t, src, dst, zeros)

    outt = pl.pallas_call(
        _dense3_body,
        grid=grid,
        in_specs=[part(2), fmaj(2), fmaj(1), full(2, 1)],
        out_specs=fmaj(2),
        out_shape=jax.ShapeDtypeStruct((2, n_pad), jnp.float32),
    )(p2, y2t, dis, b2.reshape(2, 1))

    return outt[:, :n].T


# speedup vs baseline: 185.2627x; 1.4706x over previous
import functools

import jax
import jax.numpy as jnp
from jax import lax
from jax.experimental import pallas as pl
from jax.experimental.pallas import tpu as pltpu
from jax.experimental.pallas import tpu_sc as plsc

N_SC = 2
N_TILE = 16
EDGE_CHUNK = 2000
N_SUB = 8
LANES = 16
FPAD = 8


def _sc_mesh():
    return plsc.VectorSubcoreMesh(core_axis_name="c", subcore_axis_name="s")


_SC_PARAMS = pltpu.CompilerParams(use_tc_tiling_on_sc=False,
                                  needs_layout_passes=False)


def _make_degree_kernel(n_pad, n_edges):
    per_tile = n_edges // (N_SC * N_TILE)
    n_chunks = per_tile // EDGE_CHUNK
    rows_pt = n_pad // N_TILE

    @functools.partial(
        pl.kernel,
        out_type=jax.ShapeDtypeStruct((N_SC, n_pad), jnp.float32),
        mesh=_sc_mesh(),
        compiler_params=_SC_PARAMS,
        scratch_types=[
            pltpu.VMEM_SHARED((n_pad,), jnp.float32),
            pltpu.VMEM((EDGE_CHUNK,), jnp.int32),
            pltpu.VMEM((EDGE_CHUNK,), jnp.float32),
        ],
    )
    def deg_kernel(dst_hbm, ones_hbm, zeros_hbm, out_hbm, accum, idx_d, ones_v):
        c = lax.axis_index("c")
        s = lax.axis_index("s")
        sl = pl.ds(s * rows_pt, rows_pt)
        pltpu.sync_copy(zeros_hbm.at[sl], accum.at[sl])
        pltpu.sync_copy(ones_hbm, ones_v)
        plsc.subcore_barrier()
        base = c * (n_edges // N_SC) + s * per_tile

        def body(i, carry):
            pltpu.sync_copy(dst_hbm.at[pl.ds(base + i * EDGE_CHUNK, EDGE_CHUNK)], idx_d)
            pltpu.sync_copy(ones_v, accum.at[idx_d], add=True)
            return carry

        lax.fori_loop(0, n_chunks, body, 0)
        plsc.subcore_barrier()
        pltpu.sync_copy(accum.at[sl], out_hbm.at[c, sl])

    return deg_kernel


def _make_prop_kernel(n_pad, n_edges, feat):
    per_tile = n_edges // (N_SC * N_TILE)
    n_chunks = per_tile // EDGE_CHUNK
    rows_pt = n_pad // N_TILE
    sub = rows_pt // N_SUB
    n_groups = sub // LANES

    @functools.partial(
        pl.kernel,
        out_type=jax.ShapeDtypeStruct((N_SC, feat, n_pad), jnp.float32),
        mesh=_sc_mesh(),
        compiler_params=_SC_PARAMS,
        scratch_types=[
            pltpu.VMEM_SHARED((n_pad, FPAD), jnp.float32),
            pltpu.VMEM_SHARED((n_pad, FPAD), jnp.float32),
            pltpu.VMEM((EDGE_CHUNK,), jnp.int32),
            pltpu.VMEM((EDGE_CHUNK,), jnp.int32),
            pltpu.VMEM((EDGE_CHUNK, FPAD), jnp.float32),
            pltpu.VMEM((sub,), jnp.float32),
            pltpu.VMEM((sub, FPAD), jnp.float32),
            pltpu.SemaphoreType.DMA,
        ],
    )
    def prop_kernel(yt_hbm, src_hbm, dst_hbm, out_hbm,
                    table, accum, idx_s, idx_d, rows, col_v, buf2d, sem):
        c = lax.axis_index("c")
        s = lax.axis_index("s")

        for b in range(N_SUB):
            bsl = pl.ds(s * rows_pt + b * sub, sub)
            for k in range(feat):
                pltpu.sync_copy(yt_hbm.at[k, bsl], col_v)
                kvec = jnp.full((LANES,), k, jnp.int32)

                def jbody(j, carry, kvec=kvec):
                    r = lax.iota(jnp.int32, LANES) + j * LANES
                    plsc.store_scatter(buf2d, [r, kvec], col_v[pl.ds(j * LANES, LANES)])
                    return carry

                lax.fori_loop(0, n_groups, jbody, 0)
            pltpu.sync_copy(buf2d, table.at[bsl])
            pltpu.sync_copy(buf2d, accum.at[bsl])
        plsc.subcore_barrier()
        base = c * (n_edges // N_SC) + s * per_tile

        def body(i, carry):
            e0 = base + i * EDGE_CHUNK
            pltpu.sync_copy(src_hbm.at[pl.ds(e0, EDGE_CHUNK)], idx_s)
            pltpu.sync_copy(dst_hbm.at[pl.ds(e0, EDGE_CHUNK)], idx_d)
            pltpu.async_copy(table.at[idx_s], rows, sem).wait()
            pltpu.sync_copy(rows, accum.at[idx_d], add=True)
            return carry

        lax.fori_loop(0, n_chunks, body, 0)
        plsc.subcore_barrier()

        for b in range(N_SUB):
            bsl = pl.ds(s * rows_pt + b * sub, sub)
            pltpu.sync_copy(accum.at[bsl], buf2d)
            for k in range(feat):
                kvec = jnp.full((LANES,), k, jnp.int32)

                def jbody2(j, carry, kvec=kvec):
                    r = lax.iota(jnp.int32, LANES) + j * LANES
                    col_v[pl.ds(j * LANES, LANES)] = plsc.load_gather(buf2d, [r, kvec])
                    return carry

                lax.fori_loop(0, n_groups, jbody2, 0)
                pltpu.sync_copy(col_v, out_hbm.at[c, k, bsl])

    return prop_kernel


def _dense1_body(degp_ref, xt_ref, dis_ref, y1t_ref):
    deg = degp_ref[0:1, :] + degp_ref[1:2, :] + 1.0
    dis = lax.rsqrt(deg)
    dis_ref[...] = dis
    y1t_ref[...] = xt_ref[...] * dis


def _dense2_body(p_ref, y1t_ref, dis_ref, w1t_ref, b1_ref, w2t_ref, y2t_ref):
    dis = dis_ref[...]
    t = (p_ref[0] + p_ref[1] - y1t_ref[...]) * dis
    h = jnp.dot(w1t_ref[...], t, preferred_element_type=jnp.float32) + b1_ref[...]
    h = jnp.maximum(h, 0.0)
    y2t_ref[...] = jnp.dot(w2t_ref[...], h, preferred_element_type=jnp.float32) * dis


def _dense3_body(p_ref, y2t_ref, dis_ref, b2_ref, out_ref):
    out_ref[...] = (p_ref[0] + p_ref[1] - y2t_ref[...]) * dis_ref[...] + b2_ref[...]


def kernel(x, edge_index, W1, b1, W2, b2):
    n = x.shape[0]
    n_edges = edge_index.shape[1]
    blk = -(-n // (N_TILE * 128)) * 128
    n_pad = blk * N_TILE
    pad = n_pad - n

    ei = edge_index.astype(jnp.int32)
    src = ei[0]
    dst = ei[1]
    xt = jnp.concatenate([x, jnp.zeros((pad, x.shape[1]), x.dtype)], axis=0).T

    zeros = jnp.zeros((n_pad,), jnp.float32)
    ones = jnp.ones((EDGE_CHUNK,), jnp.float32)

    degp = _make_degree_kernel(n_pad, n_edges)(dst, ones, zeros)

    grid = (N_TILE,)
    fmaj = lambda f: pl.BlockSpec((f, blk), lambda i: (0, i))
    part = lambda f: pl.BlockSpec((N_SC, f, blk), lambda i: (0, 0, i))
    full = lambda a, b: pl.BlockSpec((a, b), lambda i: (0, 0))

    dis, y1t = pl.pallas_call(
        _dense1_body,
        grid=grid,
        in_specs=[pl.BlockSpec((N_SC, blk), lambda i: (0, i)), fmaj(4)],
        out_specs=[fmaj(1), fmaj(4)],
        out_shape=[
            jax.ShapeDtypeStruct((1, n_pad), jnp.float32),
            jax.ShapeDtypeStruct((4, n_pad), jnp.float32),
        ],
    )(degp, xt)

    p1 = _make_prop_kernel(n_pad, n_edges, 4)(y1t, src, dst)

    y2t = pl.pallas_call(
        _dense2_body,
        grid=grid,
        in_specs=[part(4), fmaj(4), fmaj(1),
                  full(16, 4), full(16, 1), full(2, 16)],
        out_specs=fmaj(2),
        out_shape=jax.ShapeDtypeStruct((2, n_pad), jnp.float32),
    )(p1, y1t, dis, W1.T, b1.reshape(16, 1), W2.T)

    p2 = _make_prop_kernel(n_pad, n_edges, 2)(y2t, src, dst)

    outt = pl.pallas_call(
        _dense3_body,
        grid=grid,
        in_specs=[part(2), fmaj(2), fmaj(1), full(2, 1)],
        out_specs=fmaj(2),
        out_shape=jax.ShapeDtypeStruct((2, n_pad), jnp.float32),
    )(p2, y2t, dis, b2.reshape(2, 1))

    return outt[:, :n].T


# speedup vs baseline: 259.5568x; 1.4010x over previous
import functools

import jax
import jax.numpy as jnp
from jax import lax
from jax.experimental import pallas as pl
from jax.experimental.pallas import tpu as pltpu
from jax.experimental.pallas import tpu_sc as plsc

N_SC = 2
N_TILE = 16
EDGE_CHUNK = 1000
DEG_CHUNK = 5000
N_SUB = 8
LANES = 16
FPAD = 8


def _sc_mesh():
    return plsc.VectorSubcoreMesh(core_axis_name="c", subcore_axis_name="s")


_SC_PARAMS = pltpu.CompilerParams(use_tc_tiling_on_sc=False,
                                  needs_layout_passes=False)


def _make_degree_kernel(n_pad, n_edges):
    per_tile = n_edges // (N_SC * N_TILE)
    n_pairs = per_tile // (2 * DEG_CHUNK)
    rows_pt = n_pad // N_TILE

    @functools.partial(
        pl.kernel,
        out_type=jax.ShapeDtypeStruct((N_SC, n_pad), jnp.float32),
        mesh=_sc_mesh(),
        compiler_params=_SC_PARAMS,
        scratch_types=[
            pltpu.VMEM_SHARED((n_pad,), jnp.float32),
            pltpu.VMEM((DEG_CHUNK,), jnp.int32),
            pltpu.VMEM((DEG_CHUNK,), jnp.int32),
            pltpu.VMEM((DEG_CHUNK,), jnp.float32),
            pltpu.SemaphoreType.DMA,
            pltpu.SemaphoreType.DMA,
            pltpu.SemaphoreType.DMA,
            pltpu.SemaphoreType.DMA,
        ],
    )
    def deg_kernel(dst_hbm, ones_hbm, zeros_hbm, out_hbm,
                   accum, idx0, idx1, ones_v, si0, si1, sc0, sc1):
        c = lax.axis_index("c")
        s = lax.axis_index("s")
        sl = pl.ds(s * rows_pt, rows_pt)
        pltpu.sync_copy(zeros_hbm.at[sl], accum.at[sl])
        pltpu.sync_copy(ones_hbm, ones_v)
        plsc.subcore_barrier()
        base = c * (n_edges // N_SC) + s * per_tile

        def chunk(i):
            return dst_hbm.at[pl.ds(base + i * DEG_CHUNK, DEG_CHUNK)]

        pltpu.async_copy(chunk(0), idx0, si0)

        def body(j, carry):
            a = 2 * j

            @pl.when(j > 0)
            def _():
                pltpu.make_async_copy(ones_v, accum.at[idx1], sc1).wait()

            pltpu.async_copy(chunk(a + 1), idx1, si1)
            pltpu.make_async_copy(chunk(a), idx0, si0).wait()
            pltpu.async_copy(ones_v, accum.at[idx0], sc0, add=True)
            pltpu.make_async_copy(chunk(a + 1), idx1, si1).wait()

            @pl.when(j < n_pairs - 1)
            def _():
                pltpu.make_async_copy(ones_v, accum.at[idx0], sc0).wait()
                pltpu.async_copy(chunk(a + 2), idx0, si0)

            pltpu.async_copy(ones_v, accum.at[idx1], sc1, add=True)
            return carry

        lax.fori_loop(0, n_pairs, body, 0)
        pltpu.make_async_copy(ones_v, accum.at[idx0], sc0).wait()
        pltpu.make_async_copy(ones_v, accum.at[idx1], sc1).wait()
        plsc.subcore_barrier()
        pltpu.sync_copy(accum.at[sl], out_hbm.at[c, sl])

    return deg_kernel


def _make_prop_kernel(n_pad, n_edges, feat):
    per_tile = n_edges // (N_SC * N_TILE)
    n_pairs = per_tile // (2 * EDGE_CHUNK)
    rows_pt = n_pad // N_TILE
    sub = rows_pt // N_SUB
    n_groups = sub // LANES

    @functools.partial(
        pl.kernel,
        out_type=jax.ShapeDtypeStruct((N_SC, feat, n_pad), jnp.float32),
        mesh=_sc_mesh(),
        compiler_params=_SC_PARAMS,
        scratch_types=[
            pltpu.VMEM_SHARED((n_pad, FPAD), jnp.float32),
            pltpu.VMEM_SHARED((n_pad, FPAD), jnp.float32),
            pltpu.VMEM((EDGE_CHUNK,), jnp.int32),
            pltpu.VMEM((EDGE_CHUNK,), jnp.int32),
            pltpu.VMEM((EDGE_CHUNK,), jnp.int32),
            pltpu.VMEM((EDGE_CHUNK,), jnp.int32),
            pltpu.VMEM((EDGE_CHUNK, FPAD), jnp.float32),
            pltpu.VMEM((EDGE_CHUNK, FPAD), jnp.float32),
            pltpu.VMEM((sub,), jnp.float32),
            pltpu.VMEM((sub, FPAD), jnp.float32),
            pltpu.SemaphoreType.DMA,
            pltpu.SemaphoreType.DMA,
            pltpu.SemaphoreType.DMA,
            pltpu.SemaphoreType.DMA,
            pltpu.SemaphoreType.DMA,
            pltpu.SemaphoreType.DMA,
            pltpu.SemaphoreType.DMA,
            pltpu.SemaphoreType.DMA,
        ],
    )
    def prop_kernel(yt_hbm, src_hbm, dst_hbm, out_hbm,
                    table, accum, idx_s0, idx_s1, idx_d0, idx_d1,
                    rows0, rows1, col_v, buf2d,
                    is0, is1, id0, id1, g0, g1, sc0, sc1):
        c = lax.axis_index("c")
        s = lax.axis_index("s")

        for b in range(N_SUB):
            bsl = pl.ds(s * rows_pt + b * sub, sub)
            for k in range(feat):
                pltpu.sync_copy(yt_hbm.at[k, bsl], col_v)
                kvec = jnp.full((LANES,), k, jnp.int32)

                def jbody(j, carry, kvec=kvec):
                    r = lax.iota(jnp.int32, LANES) + j * LANES
                    plsc.store_scatter(buf2d, [r, kvec], col_v[pl.ds(j * LANES, LANES)])
                    return carry

                lax.fori_loop(0, n_groups, jbody, 0)
            pltpu.sync_copy(buf2d, table.at[bsl])
            pltpu.sync_copy(buf2d, accum.at[bsl])
        plsc.subcore_barrier()
        base = c * (n_edges // N_SC) + s * per_tile

        def srcc(i):
            return src_hbm.at[pl.ds(base + i * EDGE_CHUNK, EDGE_CHUNK)]

        def dstc(i):
            return dst_hbm.at[pl.ds(base + i * EDGE_CHUNK, EDGE_CHUNK)]

        pltpu.async_copy(srcc(0), idx_s0, is0)
        pltpu.async_copy(dstc(0), idx_d0, id0)

        def body(j, carry):
            a = 2 * j
            pltpu.make_async_copy(srcc(a), idx_s0, is0).wait()
            pltpu.async_copy(table.at[idx_s0], rows0, g0)

            @pl.when(j > 0)
            def _():
                pltpu.make_async_copy(rows1, accum.at[idx_d1], sc1).wait()

            pltpu.async_copy(srcc(a + 1), idx_s1, is1)
            pltpu.async_copy(dstc(a + 1), idx_d1, id1)
            pltpu.make_async_copy(table.at[idx_s0], rows0, g0).wait()
            pltpu.make_async_copy(dstc(a), idx_d0, id0).wait()
            pltpu.async_copy(rows0, accum.at[idx_d0], sc0, add=True)
            pltpu.make_async_copy(srcc(a + 1), idx_s1, is1).wait()
            pltpu.async_copy(table.at[idx_s1], rows1, g1)

            @pl.when(j < n_pairs - 1)
            def _():
                pltpu.make_async_copy(rows0, accum.at[idx_d0], sc0).wait()
                pltpu.async_copy(srcc(a + 2), idx_s0, is0)
                pltpu.async_copy(dstc(a + 2), idx_d0, id0)

            pltpu.make_async_copy(table.at[idx_s1], rows1, g1).wait()
            pltpu.make_async_copy(dstc(a + 1), idx_d1, id1).wait()
            pltpu.async_copy(rows1, accum.at[idx_d1], sc1, add=True)
            return carry

        lax.fori_loop(0, n_pairs, body, 0)
        pltpu.make_async_copy(rows0, accum.at[idx_d0], sc0).wait()
        pltpu.make_async_copy(rows1, accum.at[idx_d1], sc1).wait()
        plsc.subcore_barrier()

        for b in range(N_SUB):
            bsl = pl.ds(s * rows_pt + b * sub, sub)
            pltpu.sync_copy(accum.at[bsl], buf2d)
            for k in range(feat):
                kvec = jnp.full((LANES,), k, jnp.int32)

                def jbody2(j, carry, kvec=kvec):
                    r = lax.iota(jnp.int32, LANES) + j * LANES
                    col_v[pl.ds(j * LANES, LANES)] = plsc.load_gather(buf2d, [r, kvec])
                    return carry

                lax.fori_loop(0, n_groups, jbody2, 0)
                pltpu.sync_copy(col_v, out_hbm.at[c, k, bsl])

    return prop_kernel


def _dense1_body(degp_ref, xt_ref, dis_ref, y1t_ref):
    deg = degp_ref[0:1, :] + degp_ref[1:2, :] + 1.0
    dis = lax.rsqrt(deg)
    dis_ref[...] = dis
    y1t_ref[...] = xt_ref[...] * dis


def _dense2_body(p_ref, y1t_ref, dis_ref, w1t_ref, b1_ref, w2t_ref, y2t_ref):
    dis = dis_ref[...]
    t = (p_ref[0] + p_ref[1] - y1t_ref[...]) * dis
    h = jnp.dot(w1t_ref[...], t, preferred_element_type=jnp.float32) + b1_ref[...]
    h = jnp.maximum(h, 0.0)
    y2t_ref[...] = jnp.dot(w2t_ref[...], h, preferred_element_type=jnp.float32) * dis


def _dense3_body(p_ref, y2t_ref, dis_ref, b2_ref, out_ref):
    out_ref[...] = (p_ref[0] + p_ref[1] - y2t_ref[...]) * dis_ref[...] + b2_ref[...]


def kernel(x, edge_index, W1, b1, W2, b2):
    n = x.shape[0]
    n_edges = edge_index.shape[1]
    blk = -(-n // (N_TILE * 128)) * 128
    n_pad = blk * N_TILE
    pad = n_pad - n

    ei = edge_index.astype(jnp.int32)
    src = ei[0]
    dst = ei[1]
    xt = jnp.concatenate([x, jnp.zeros((pad, x.shape[1]), x.dtype)], axis=0).T

    zeros = jnp.zeros((n_pad,), jnp.float32)
    ones = jnp.ones((DEG_CHUNK,), jnp.float32)

    degp = _make_degree_kernel(n_pad, n_edges)(dst, ones, zeros)

    grid = (N_TILE,)
    fmaj = lambda f: pl.BlockSpec((f, blk), lambda i: (0, i))
    part = lambda f: pl.BlockSpec((N_SC, f, blk), lambda i: (0, 0, i))
    full = lambda a, b: pl.BlockSpec((a, b), lambda i: (0, 0))

    dis, y1t = pl.pallas_call(
        _dense1_body,
        grid=grid,
        in_specs=[pl.BlockSpec((N_SC, blk), lambda i: (0, i)), fmaj(4)],
        out_specs=[fmaj(1), fmaj(4)],
        out_shape=[
            jax.ShapeDtypeStruct((1, n_pad), jnp.float32),
            jax.ShapeDtypeStruct((4, n_pad), jnp.float32),
        ],
    )(degp, xt)

    p1 = _make_prop_kernel(n_pad, n_edges, 4)(y1t, src, dst)

    y2t = pl.pallas_call(
        _dense2_body,
        grid=grid,
        in_specs=[part(4), fmaj(4), fmaj(1),
                  full(16, 4), full(16, 1), full(2, 16)],
        out_specs=fmaj(2),
        out_shape=jax.ShapeDtypeStruct((2, n_pad), jnp.float32),
    )(p1, y1t, dis, W1.T, b1.reshape(16, 1), W2.T)

    p2 = _make_prop_kernel(n_pad, n_edges, 2)(y2t, src, dst)

    outt = pl.pallas_call(
        _dense3_body,
        grid=grid,
        in_specs=[part(2), fmaj(2), fmaj(1), full(2, 1)],
        out_specs=fmaj(2),
        out_shape=jax.ShapeDtypeStruct((2, n_pad), jnp.float32),
    )(p2, y2t, dis, b2.reshape(2, 1))

    return outt[:, :n].T
